# baseline (speedup 1.0000x reference)
import functools

import jax
import jax.numpy as jnp
from jax import lax
from jax.experimental import pallas as pl
from jax.experimental.pallas import tpu as pltpu
from jax.experimental.pallas import tpu_sc as plsc

E = 3_200_000
DIM = 16
NC, NS = 2, 16
NW = NC * NS
PER_W = E // NW
SG = 80
K = 10
CHUNK = SG * K
NCHUNK = PER_W // CHUNK

_mesh = plsc.VectorSubcoreMesh(core_axis_name="c", subcore_axis_name="s")


@functools.partial(
    pl.kernel,
    out_type=jax.ShapeDtypeStruct((E, DIM), jnp.float32),
    mesh=_mesh,
    scratch_types=[
        pltpu.VMEM((CHUNK,), jnp.int32),
        pltpu.VMEM((CHUNK, DIM), jnp.float32),
        pltpu.SemaphoreType.DMA,
    ],
    compiler_params=pltpu.CompilerParams(use_tc_tiling_on_sc=False),
)
def _embed(mask_hbm, table_hbm, out_hbm, idx_v, rows_v, sem):
    wid = lax.axis_index("s") * NC + lax.axis_index("c")
    base = wid * PER_W

    @pl.loop(0, NCHUNK)
    def _chunk(i):
        off = base + i * CHUNK
        pltpu.sync_copy(mask_hbm.at[pl.ds(off, CHUNK)], idx_v)
        copies = [
            pltpu.async_copy(
                table_hbm.at[idx_v.at[pl.ds(j * SG, SG)]],
                rows_v.at[pl.ds(j * SG, SG)],
                sem,
            )
            for j in range(K)
        ]
        for c in copies:
            c.wait()
        pltpu.sync_copy(rows_v, out_hbm.at[pl.ds(off, CHUNK)])


def kernel(bond_mask, table):
    return _embed(bond_mask, table)


# speedup vs baseline: 8.9809x; 8.9809x over previous
import functools

import jax
import jax.numpy as jnp
from jax import lax
from jax.experimental import pallas as pl
from jax.experimental.pallas import tpu as pltpu
from jax.experimental.pallas import tpu_sc as plsc

E = 3_200_000
DIM = 16
NC, NS = 2, 16
NW = NC * NS
PER_W = E // NW
CHUNK = 2_000
NCHUNK = PER_W // CHUNK
GROUPS = CHUNK // DIM

_mesh = plsc.VectorSubcoreMesh(core_axis_name="c", subcore_axis_name="s")


@functools.partial(
    pl.kernel,
    out_type=jax.ShapeDtypeStruct((E * DIM,), jnp.float32),
    mesh=_mesh,
    scratch_types=[
        pltpu.VMEM((2 * DIM,), jnp.float32),
        pltpu.VMEM((2, CHUNK), jnp.int32),
        pltpu.VMEM((2, CHUNK * DIM), jnp.float32),
        pltpu.SemaphoreType.DMA,
        pltpu.SemaphoreType.DMA,
    ],
    compiler_params=pltpu.CompilerParams(
        use_tc_tiling_on_sc=False, needs_layout_passes=False),
)
def _embed(mask_hbm, table_hbm, out_hbm, table_v, mask_v, rows_v, sem_in, sem_out):
    wid = lax.axis_index("s") * NC + lax.axis_index("c")
    base = wid * PER_W
    pltpu.sync_copy(table_hbm, table_v)
    iota = lax.broadcasted_iota(jnp.int32, (16,), 0)
    iota16 = iota * DIM

    def in_copy(i, slot):
        off = base + i * CHUNK
        return pltpu.async_copy(
            mask_hbm.at[pl.ds(off, CHUNK)], mask_v.at[slot], sem_in)

    def out_copy(i, slot):
        off = (base + i * CHUNK) * DIM
        return pltpu.async_copy(
            rows_v.at[slot], out_hbm.at[pl.ds(off, CHUNK * DIM)], sem_out)

    def wait_in(slot):
        pltpu.make_async_copy(
            mask_hbm.at[pl.ds(base, CHUNK)], mask_v.at[slot], sem_in).wait()

    def wait_out(slot):
        pltpu.make_async_copy(
            rows_v.at[slot],
            out_hbm.at[pl.ds(base * DIM, CHUNK * DIM)], sem_out).wait()

    def compute(slot):
        rows = rows_v.at[slot]

        @pl.loop(0, GROUPS)
        def _group(g):
            m = mask_v[slot, pl.ds(g * DIM, DIM)]
            midx = m * DIM
            sbase = iota16 + g * (DIM * DIM)
            for l in range(DIM):
                v = plsc.load_gather(table_v, [midx + l])
                plsc.store_scatter(rows, [sbase + l], v)

    in_copy(0, 0)
    in_copy(1, 1)

    @pl.loop(0, NCHUNK, step=2)
    def _chunk(i):
        for s in (0, 1):
            ii = i + s
            wait_in(s)

            @pl.when(ii >= 2)
            def _():
                wait_out(s)

            compute(s)
            out_copy(ii, s)

            @pl.when(ii + 2 < NCHUNK)
            def _():
                in_copy(ii + 2, s)

    wait_out(0)
    wait_out(1)


def kernel(bond_mask, table):
    return _embed(bond_mask, jnp.reshape(table, (2 * DIM,))).reshape(E, DIM)
